# Initial kernel scaffold; baseline (speedup 1.0000x reference)
import jax
import jax.numpy as jnp
from jax.experimental import pallas as pl


def kernel(indices, tables, W1, b1, W2, b2, W3, b3):
    raise NotImplementedError("write your pallas kernel here")


# speedup vs baseline: 7.0165x; 7.0165x over previous
import functools

import jax
import jax.numpy as jnp
import numpy as np
from jax import lax
from jax.experimental import pallas as pl
from jax.experimental.pallas import tpu as pltpu
from jax.experimental.pallas import tpu_sc as plsc

NW = 32
GATHER_N = 128
CHUNK_ROWS = 1664


def _sc_gather(tab2, idx_flat, offsets, n_rows, d):
    n_per_w = n_rows // NW
    n_chunks = n_per_w // CHUNK_ROWS
    n_g = CHUNK_ROWS // GATHER_N
    mesh = plsc.VectorSubcoreMesh(core_axis_name="c", subcore_axis_name="s")

    @functools.partial(
        pl.kernel, mesh=mesh,
        out_type=jax.ShapeDtypeStruct((n_rows, d), jnp.float32),
        compiler_params=pltpu.CompilerParams(use_tc_tiling_on_sc=False),
        scratch_types=[
            pltpu.VMEM((n_per_w,), jnp.int32),
            pltpu.VMEM((n_per_w,), jnp.int32),
            pltpu.VMEM((CHUNK_ROWS, d), jnp.float32),
            pltpu.SemaphoreType.DMA,
        ],
    )
    def k(tab_hbm, idx_hbm, off_hbm, out_hbm, idx_v, off_v, rows_v, sem):
        wid = lax.axis_index("s") * 2 + lax.axis_index("c")
        base = wid * n_per_w
        pltpu.sync_copy(idx_hbm.at[pl.ds(base, n_per_w)], idx_v)
        pltpu.sync_copy(off_hbm, off_v)

        def add_body(i, carry):
            sl = pl.ds(i * 16, 16)
            idx_v[sl] = idx_v[sl] + off_v[sl]
            return carry

        lax.fori_loop(0, n_per_w // 16, add_body, 0)

        def chunk_body(c, carry):
            row0 = c * CHUNK_ROWS
            cps = [
                pltpu.async_copy(
                    tab_hbm.at[idx_v.at[pl.ds(row0 + g * GATHER_N, GATHER_N)]],
                    rows_v.at[pl.ds(g * GATHER_N, GATHER_N)],
                    sem,
                )
                for g in range(n_g)
            ]
            for cp in cps:
                cp.wait()
            pltpu.sync_copy(rows_v, out_hbm.at[pl.ds(base + row0, CHUNK_ROWS)])
            return carry

        lax.fori_loop(0, n_chunks, chunk_body, 0)

    return k(tab2, idx_flat, offsets)


def _tc_mlp(ft3, w1at, w1bft, b1c, w2t, b2c, w3t, b3c, bb):
    f, d, b = ft3.shape
    h1 = w1at.shape[0]
    h2 = w2t.shape[0]

    def body(ft_ref, w1a_ref, w1b_ref, b1_ref, w2_ref, b2_ref, w3_ref,
             b3_ref, out_ref):
        ft = ft_ref[...]
        flat_t = ft.reshape(f * d, bb)
        gs = []
        for i in range(f):
            prod = ft * ft[i][None]
            gs.append(jnp.sum(prod, axis=1))
        gt = jnp.concatenate(gs, axis=0)
        h = jnp.dot(w1a_ref[...], flat_t, preferred_element_type=jnp.float32)
        h = h + jnp.dot(w1b_ref[...], gt, preferred_element_type=jnp.float32)
        h = jnp.maximum(h + b1_ref[...], 0.0)
        h = jnp.dot(w2_ref[...], h, preferred_element_type=jnp.float32)
        h = jnp.maximum(h + b2_ref[...], 0.0)
        o = jnp.dot(w3_ref[...], h, preferred_element_type=jnp.float32)
        out_ref[...] = o + b3_ref[...]

    return pl.pallas_call(
        body,
        grid=(b // bb,),
        in_specs=[
            pl.BlockSpec((f, d, bb), lambda i: (0, 0, i)),
            pl.BlockSpec((h1, f * d), lambda i: (0, 0)),
            pl.BlockSpec((h1, f * f), lambda i: (0, 0)),
            pl.BlockSpec((h1, 1), lambda i: (0, 0)),
            pl.BlockSpec((h2, h1), lambda i: (0, 0)),
            pl.BlockSpec((h2, 1), lambda i: (0, 0)),
            pl.BlockSpec((1, h2), lambda i: (0, 0)),
            pl.BlockSpec((1, 1), lambda i: (0, 0)),
        ],
        out_specs=pl.BlockSpec((1, bb), lambda i: (0, i)),
        out_shape=jax.ShapeDtypeStruct((1, b), jnp.float32),
    )(ft3, w1at, w1bft, b1c, w2t, b2c, w3t, b3c)


def kernel(indices, tables, W1, b1, W2, b2, W3, b3):
    bsz, f = indices.shape
    _, v, d = tables.shape
    h1 = W1.shape[1]

    tab2 = tables.reshape(f * v, d)
    idx_flat = indices.reshape(bsz * f)
    n_per_w = bsz * f // NW
    offsets = jnp.tile(jnp.arange(f, dtype=jnp.int32) * v, n_per_w // f)

    feats_rows = _sc_gather(tab2, idx_flat, offsets, bsz * f, d)
    ft3 = feats_rows.reshape(bsz, f, d).transpose(1, 2, 0)

    iu, ju = np.triu_indices(f, k=1)
    w1a = W1[: f * d]
    w1bf = jnp.zeros((f * f, h1), W1.dtype).at[iu * f + ju].set(W1[f * d:])

    out2 = _tc_mlp(ft3, w1a.T, w1bf.T, b1[:, None], W2.T, b2[:, None],
                   W3.T, b3[None, :], 512)
    return out2.reshape(bsz)


# speedup vs baseline: 8.0331x; 1.1449x over previous
import functools

import jax
import jax.numpy as jnp
import numpy as np
from jax import lax
from jax.experimental import pallas as pl
from jax.experimental.pallas import tpu as pltpu
from jax.experimental.pallas import tpu_sc as plsc

NW = 32
GATHER_N = 128
CHUNK_ROWS = 1664


def _sc_gather(tab2, idx_flat, offsets, n_rows, d):
    n_per_w = n_rows // NW
    n_chunks = n_per_w // CHUNK_ROWS
    n_g = CHUNK_ROWS // GATHER_N
    mesh = plsc.VectorSubcoreMesh(core_axis_name="c", subcore_axis_name="s")

    @functools.partial(
        pl.kernel, mesh=mesh,
        out_type=jax.ShapeDtypeStruct((n_rows, d), jnp.float32),
        compiler_params=pltpu.CompilerParams(use_tc_tiling_on_sc=False),
        scratch_types=[
            pltpu.VMEM((n_per_w,), jnp.int32),
            pltpu.VMEM((n_per_w,), jnp.int32),
            pltpu.VMEM((CHUNK_ROWS, d), jnp.float32),
            pltpu.SemaphoreType.DMA,
        ],
    )
    def k(tab_hbm, idx_hbm, off_hbm, out_hbm, idx_v, off_v, rows_v, sem):
        wid = lax.axis_index("s") * 2 + lax.axis_index("c")
        base = wid * n_per_w
        pltpu.sync_copy(idx_hbm.at[pl.ds(base, n_per_w)], idx_v)
        pltpu.sync_copy(off_hbm, off_v)

        def add_body(i, carry):
            sl = pl.ds(i * 16, 16)
            idx_v[sl] = idx_v[sl] + off_v[sl]
            return carry

        lax.fori_loop(0, n_per_w // 16, add_body, 0)

        def chunk_body(c, carry):
            row0 = c * CHUNK_ROWS
            cps = [
                pltpu.async_copy(
                    tab_hbm.at[idx_v.at[pl.ds(row0 + g * GATHER_N, GATHER_N)]],
                    rows_v.at[pl.ds(g * GATHER_N, GATHER_N)],
                    sem,
                )
                for g in range(n_g)
            ]
            for cp in cps:
                cp.wait()
            pltpu.sync_copy(rows_v, out_hbm.at[pl.ds(base + row0, CHUNK_ROWS)])
            return carry

        lax.fori_loop(0, n_chunks, chunk_body, 0)

    return k(tab2, idx_flat, offsets)


def _tc_mlp(flat2, w1at, w1bft, b1c, w2t, b2c, w3t, b3c, bb, f, d):
    b = flat2.shape[0]
    h1 = w1at.shape[0]
    h2 = w2t.shape[0]

    def body(flat_ref, w1a_ref, w1b_ref, b1_ref, w2_ref, b2_ref, w3_ref,
             b3_ref, out_ref):
        flat_t = jnp.transpose(flat_ref[...])
        ft = flat_t.reshape(f, d, bb)
        gs = []
        for i in range(f):
            prod = ft * ft[i][None]
            gs.append(jnp.sum(prod, axis=1))
        gt = jnp.concatenate(gs, axis=0)
        h = jnp.dot(w1a_ref[...], flat_t, preferred_element_type=jnp.float32)
        h = h + jnp.dot(w1b_ref[...], gt, preferred_element_type=jnp.float32)
        h = jnp.maximum(h + b1_ref[...], 0.0)
        h = jnp.dot(w2_ref[...], h, preferred_element_type=jnp.float32)
        h = jnp.maximum(h + b2_ref[...], 0.0)
        o = jnp.dot(w3_ref[...], h, preferred_element_type=jnp.float32)
        out_ref[...] = o + b3_ref[...]

    return pl.pallas_call(
        body,
        grid=(b // bb,),
        in_specs=[
            pl.BlockSpec((bb, f * d), lambda i: (i, 0)),
            pl.BlockSpec((h1, f * d), lambda i: (0, 0)),
            pl.BlockSpec((h1, f * f), lambda i: (0, 0)),
            pl.BlockSpec((h1, 1), lambda i: (0, 0)),
            pl.BlockSpec((h2, h1), lambda i: (0, 0)),
            pl.BlockSpec((h2, 1), lambda i: (0, 0)),
            pl.BlockSpec((1, h2), lambda i: (0, 0)),
            pl.BlockSpec((1, 1), lambda i: (0, 0)),
        ],
        out_specs=pl.BlockSpec((1, bb), lambda i: (0, i)),
        out_shape=jax.ShapeDtypeStruct((1, b), jnp.float32),
    )(flat2, w1at, w1bft, b1c, w2t, b2c, w3t, b3c)


def kernel(indices, tables, W1, b1, W2, b2, W3, b3):
    bsz, f = indices.shape
    _, v, d = tables.shape
    h1 = W1.shape[1]

    tab2 = tables.reshape(f * v, d)
    idx_flat = indices.reshape(bsz * f)
    n_per_w = bsz * f // NW
    offsets = jnp.tile(jnp.arange(f, dtype=jnp.int32) * v, n_per_w // f)

    feats_rows = _sc_gather(tab2, idx_flat, offsets, bsz * f, d)
    flat2 = feats_rows.reshape(bsz, f * d)

    iu, ju = np.triu_indices(f, k=1)
    w1a = W1[: f * d]
    w1bf = jnp.zeros((f * f, h1), W1.dtype).at[iu * f + ju].set(W1[f * d:])

    out2 = _tc_mlp(flat2, w1a.T, w1bf.T, b1[:, None], W2.T, b2[:, None],
                   W3.T, b3[None, :], 512, f, d)
    return out2.reshape(bsz)
